# initial kernel scaffold (unmeasured)
import jax
import jax.numpy as jnp
from jax import lax
from jax.experimental import pallas as pl
from jax.experimental.pallas import tpu as pltpu


def kernel(
    x,
):
    def body(*refs):
        pass

    out_shape = jax.ShapeDtypeStruct(..., jnp.float32)
    return pl.pallas_call(body, out_shape=out_shape)(...)


# baseline (device time: 32061 ns/iter reference)
import jax
import jax.numpy as jnp
from jax import lax
from jax.experimental import pallas as pl
from jax.experimental.pallas import tpu as pltpu

N_DEV = 4


def kernel(x):
    m_per, n = x.shape
    ch = m_per // N_DEV

    def body(x_ref, out_ref, rs_send, rs_recv, ag_send0, ag_recv,
             rs_send_sems, rs_recv_sems, ag_send_sems, ag_recv_sems):
        my = lax.axis_index("i")
        left = lax.rem(my + N_DEV - 1, N_DEV)
        right = lax.rem(my + 1, N_DEV)

        barrier_sem = pltpu.get_barrier_semaphore()
        for nbr in (left, right):
            pl.semaphore_signal(
                barrier_sem, inc=1,
                device_id=(nbr,), device_id_type=pl.DeviceIdType.MESH,
            )
        pl.semaphore_wait(barrier_sem, 2)

        def chunk(c):
            return x_ref[pl.ds(c * ch, ch), :].astype(jnp.bfloat16)

        rs_send[0, :, :] = chunk(my)
        for s in range(N_DEV - 1):
            rdma = pltpu.make_async_remote_copy(
                src_ref=rs_send.at[s],
                dst_ref=rs_recv.at[s],
                send_sem=rs_send_sems.at[s],
                recv_sem=rs_recv_sems.at[s],
                device_id=(right,),
                device_id_type=pl.DeviceIdType.MESH,
            )
            rdma.start()
            rdma.wait()
            c = lax.rem(my + N_DEV - s - 1, N_DEV)
            acc = rs_recv[s, :, :] + chunk(c)
            if s < N_DEV - 2:
                rs_send[s + 1, :, :] = acc
            else:
                ag_send0[:, :] = acc
                out_ref[pl.ds(right * ch, ch), :] = acc

        for t in range(N_DEV - 1):
            src = ag_send0 if t == 0 else ag_recv.at[t - 1]
            rdma = pltpu.make_async_remote_copy(
                src_ref=src,
                dst_ref=ag_recv.at[t],
                send_sem=ag_send_sems.at[t],
                recv_sem=ag_recv_sems.at[t],
                device_id=(right,),
                device_id_type=pl.DeviceIdType.MESH,
            )
            rdma.start()
            rdma.wait()
            origin = lax.rem(my + N_DEV - t, N_DEV)
            out_ref[pl.ds(origin * ch, ch), :] = ag_recv[t, :, :]

    return pl.pallas_call(
        body,
        out_shape=jax.ShapeDtypeStruct((m_per, n), jnp.bfloat16),
        in_specs=[pl.BlockSpec(memory_space=pltpu.VMEM)],
        out_specs=pl.BlockSpec(memory_space=pltpu.VMEM),
        scratch_shapes=[
            pltpu.VMEM((N_DEV - 1, ch, n), jnp.bfloat16),
            pltpu.VMEM((N_DEV - 1, ch, n), jnp.bfloat16),
            pltpu.VMEM((ch, n), jnp.bfloat16),
            pltpu.VMEM((N_DEV - 1, ch, n), jnp.bfloat16),
            pltpu.SemaphoreType.DMA((N_DEV - 1,)),
            pltpu.SemaphoreType.DMA((N_DEV - 1,)),
            pltpu.SemaphoreType.DMA((N_DEV - 1,)),
            pltpu.SemaphoreType.DMA((N_DEV - 1,)),
        ],
        compiler_params=pltpu.CompilerParams(collective_id=0),
    )(x)


# device time: 20711 ns/iter; 1.5480x vs baseline; 1.5480x over previous
import jax
import jax.numpy as jnp
from jax import lax
from jax.experimental import pallas as pl
from jax.experimental.pallas import tpu as pltpu

N_DEV = 4


def kernel(x):
    m, n = x.shape
    q = m // 4
    h = m // 2
    cw = n // 2

    CA = pl.ds(0, cw)
    CB = pl.ds(cw, cw)

    def body(x_ref, out_ref, st1a, st1b, r1a, r1b, r2a, r2b,
             send_sems, recv_sems):
        my = lax.axis_index("i")
        g = my // 2
        b = lax.rem(lax.rem(my, 2) + g, 2)
        p_b = jnp.bitwise_xor(my, 1)
        p_g = jnp.bitwise_xor(my, 3)

        def rows(k):
            return pl.ds(k * q, q)

        def bf(row_slice, col_slice):
            return x_ref[row_slice, col_slice].astype(jnp.bfloat16)

        barrier_sem = pltpu.get_barrier_semaphore()
        for nbr in (p_b, p_g):
            pl.semaphore_signal(
                barrier_sem, inc=1,
                device_id=(nbr,), device_id_type=pl.DeviceIdType.MESH,
            )
        pl.semaphore_wait(barrier_sem, 2)

        def rdma(src, dst, sem_idx, dev):
            return pltpu.make_async_remote_copy(
                src_ref=src, dst_ref=dst,
                send_sem=send_sems.at[sem_idx],
                recv_sem=recv_sems.at[sem_idx],
                device_id=(dev,), device_id_type=pl.DeviceIdType.MESH,
            )

        ga = 1 - g
        st1a[0, :, :] = bf(rows(2 * ga), CA)
        st1a[1, :, :] = bf(rows(2 * ga + 1), CA)
        bb = 1 - b
        st1b[0, :, :] = bf(rows(bb), CB)
        st1b[1, :, :] = bf(rows(2 + bb), CB)
        c1a = rdma(st1a, r1a, 0, p_g)
        c1b = rdma(st1b, r1b, 1, p_b)
        c1a.start()
        c1b.start()
        out_ref[rows(2 * g), CA] = bf(rows(2 * g), CA)
        out_ref[rows(2 * g + 1), CA] = bf(rows(2 * g + 1), CA)
        out_ref[rows(b), CB] = bf(rows(b), CB)
        out_ref[rows(2 + b), CB] = bf(rows(2 + b), CB)
        c1a.wait()
        c1b.wait()
        out_ref[rows(2 * g), CA] = out_ref[rows(2 * g), CA] + r1a[0, :, :]
        out_ref[rows(2 * g + 1), CA] = (
            out_ref[rows(2 * g + 1), CA] + r1a[1, :, :])
        out_ref[rows(b), CB] = out_ref[rows(b), CB] + r1b[0, :, :]
        out_ref[rows(2 + b), CB] = out_ref[rows(2 + b), CB] + r1b[1, :, :]

        k_own = 2 * g + b
        c2a = rdma(out_ref.at[rows(2 * g + 1 - b), CA], r2a, 2, p_b)
        c2b = rdma(out_ref.at[rows(2 * (1 - g) + b), CB], r2b, 3, p_g)
        c2a.start()
        c2b.start()
        c2a.wait()
        c2b.wait()
        out_ref[rows(k_own), CA] = out_ref[rows(k_own), CA] + r2a[:, :]
        out_ref[rows(k_own), CB] = out_ref[rows(k_own), CB] + r2b[:, :]

        c3a = rdma(out_ref.at[rows(k_own), CA],
                   out_ref.at[rows(k_own), CA], 4, p_b)
        c3b = rdma(out_ref.at[rows(k_own), CB],
                   out_ref.at[rows(k_own), CB], 5, p_g)
        c3a.start()
        c3b.start()
        c3a.wait()
        c3b.wait()

        c4a = rdma(out_ref.at[pl.ds(g * h, h), CA],
                   out_ref.at[pl.ds(g * h, h), CA], 6, p_g)
        c4b0 = rdma(out_ref.at[rows(b), CB],
                    out_ref.at[rows(b), CB], 7, p_b)
        c4b1 = rdma(out_ref.at[rows(2 + b), CB],
                    out_ref.at[rows(2 + b), CB], 8, p_b)
        c4a.start()
        c4b0.start()
        c4b1.start()
        c4a.wait()
        c4b0.wait()
        c4b1.wait()

    return pl.pallas_call(
        body,
        out_shape=jax.ShapeDtypeStruct((m, n), jnp.bfloat16),
        in_specs=[pl.BlockSpec(memory_space=pltpu.VMEM)],
        out_specs=pl.BlockSpec(memory_space=pltpu.VMEM),
        scratch_shapes=[
            pltpu.VMEM((2, q, cw), jnp.bfloat16),
            pltpu.VMEM((2, q, cw), jnp.bfloat16),
            pltpu.VMEM((2, q, cw), jnp.bfloat16),
            pltpu.VMEM((2, q, cw), jnp.bfloat16),
            pltpu.VMEM((q, cw), jnp.bfloat16),
            pltpu.VMEM((q, cw), jnp.bfloat16),
            pltpu.SemaphoreType.DMA((9,)),
            pltpu.SemaphoreType.DMA((9,)),
        ],
        compiler_params=pltpu.CompilerParams(collective_id=0),
    )(x)
